# baseline (speedup 1.0000x reference)
import functools

import numpy as np

import jax
import jax.numpy as jnp
from jax import lax
from jax.experimental import pallas as pl
from jax.experimental.pallas import tpu as pltpu
from jax.experimental.pallas import tpu_sc as plsc

ROWS = 128
COLS = 32768
NUM_CORES = 2
NUM_SUBCORES = 16
NUM_WORKERS = NUM_CORES * NUM_SUBCORES
ROWS_PER_WORKER = ROWS // NUM_WORKERS
LANES = 16
NVECS = COLS // LANES
UNROLL = 16
NGROUPS = NVECS // UNROLL

def _vreg_prefix_shift(v, shift_consts):
    s = v
    for idx, msk in shift_consts:
        g = s.at[idx].get(mode="promise_in_bounds")
        s = s + jnp.where(msk, g, jnp.float32(0.0))
    return s


def _inclusive_prefix_tree(ts):
    n = len(ts)
    a = list(ts)
    d = 1
    while d < n:
        for start in range(0, n, 2 * d):
            left_last = a[start + d - 1]
            for j in range(start + d, min(start + 2 * d, n)):
                a[j] = a[j] + left_last
        d *= 2
    return a


def _sc_row_cumsum(x):
    mesh = plsc.VectorSubcoreMesh(
        core_axis_name="c", subcore_axis_name="s")

    @functools.partial(
        pl.kernel,
        out_type=jax.ShapeDtypeStruct((ROWS, COLS), jnp.float32),
        mesh=mesh,
        scratch_types=[
            pltpu.VMEM((2, COLS), jnp.float32),
            pltpu.SemaphoreType.DMA,
            pltpu.SemaphoreType.DMA,
            pltpu.SemaphoreType.DMA,
            pltpu.SemaphoreType.DMA,
        ],
        compiler_params=pltpu.CompilerParams(needs_layout_passes=False),
    )
    def k(x_hbm, out_hbm, buf, in_sem0, in_sem1, out_sem0, out_sem1):
        wid = lax.axis_index("s") * NUM_CORES + lax.axis_index("c")
        iota = lax.iota(jnp.int32, LANES)
        idx_last = jnp.full((LANES,), LANES - 1, jnp.int32)
        shift_consts = [(jnp.maximum(iota - d, 0), iota >= d)
                        for d in (1, 2, 4, 8)]
        in_sems = (in_sem0, in_sem1)
        out_sems = (out_sem0, out_sem1)

        def row_idx(r):
            return wid * ROWS_PER_WORKER + r

        def scan_row(b):
            def group_body(g, c):
                base = g * (UNROLL * LANES)
                sls = [pl.ds(base + j * LANES, LANES) for j in range(UNROLL)]
                ss = []
                for j in range(UNROLL):
                    v = buf[b, sls[j]]
                    ss.append(_vreg_prefix_shift(v, shift_consts)
                              if j % 3 == 2 else plsc.cumsum(v))
                ts = [s.at[idx_last].get(mode="promise_in_bounds")
                      for s in ss]
                incl = _inclusive_prefix_tree(ts)
                pres = [c] + [c + incl[j] for j in range(UNROLL - 1)]
                for j in range(UNROLL):
                    buf[b, sls[j]] = ss[j] + pres[j]
                return c + incl[UNROLL - 1]

            plsc.parallel_loop(
                0, NGROUPS, 1, carry=jnp.zeros((LANES,), jnp.float32)
            )(group_body)

        pending_out = [None, None]
        copy_in = pltpu.async_copy(
            x_hbm.at[row_idx(0)], buf.at[0], in_sems[0])
        for r in range(ROWS_PER_WORKER):
            b = r % 2
            nb = (r + 1) % 2
            if r + 1 < ROWS_PER_WORKER:
                if pending_out[nb] is not None:
                    pending_out[nb].wait()
                    pending_out[nb] = None
                next_in = pltpu.async_copy(
                    x_hbm.at[row_idx(r + 1)], buf.at[nb], in_sems[nb])
            copy_in.wait()
            scan_row(b)
            pending_out[b] = pltpu.async_copy(
                buf.at[b], out_hbm.at[row_idx(r)], out_sems[b])
            if r + 1 < ROWS_PER_WORKER:
                copy_in = next_in
        for p in pending_out:
            if p is not None:
                p.wait()

    return k(x)


def kernel(x):
    return _sc_row_cumsum(x)


# speedup vs baseline: 1.0382x; 1.0382x over previous
import functools

import numpy as np

import jax
import jax.numpy as jnp
from jax import lax
from jax.experimental import pallas as pl
from jax.experimental.pallas import tpu as pltpu
from jax.experimental.pallas import tpu_sc as plsc

ROWS = 128
COLS = 32768
NUM_CORES = 2
NUM_SUBCORES = 16
NUM_WORKERS = NUM_CORES * NUM_SUBCORES
ROWS_PER_WORKER = ROWS // NUM_WORKERS
LANES = 16
NVECS = COLS // LANES
UNROLL = 8
NGROUPS = NVECS // UNROLL

def _vreg_prefix_shift(v, shift_consts):
    s = v
    for idx, msk in shift_consts:
        g = s.at[idx].get(mode="promise_in_bounds")
        s = s + jnp.where(msk, g, jnp.float32(0.0))
    return s


def _inclusive_prefix_tree(ts):
    n = len(ts)
    a = list(ts)
    d = 1
    while d < n:
        for start in range(0, n, 2 * d):
            left_last = a[start + d - 1]
            for j in range(start + d, min(start + 2 * d, n)):
                a[j] = a[j] + left_last
        d *= 2
    return a


def _sc_row_cumsum(x):
    mesh = plsc.VectorSubcoreMesh(
        core_axis_name="c", subcore_axis_name="s")

    @functools.partial(
        pl.kernel,
        out_type=jax.ShapeDtypeStruct((ROWS, COLS), jnp.float32),
        mesh=mesh,
        scratch_types=[
            pltpu.VMEM((2, COLS), jnp.float32),
            pltpu.SemaphoreType.DMA,
            pltpu.SemaphoreType.DMA,
            pltpu.SemaphoreType.DMA,
            pltpu.SemaphoreType.DMA,
        ],
        compiler_params=pltpu.CompilerParams(needs_layout_passes=False),
    )
    def k(x_hbm, out_hbm, buf, in_sem0, in_sem1, out_sem0, out_sem1):
        wid = lax.axis_index("s") * NUM_CORES + lax.axis_index("c")
        iota = lax.iota(jnp.int32, LANES)
        idx_last = jnp.full((LANES,), LANES - 1, jnp.int32)
        shift_consts = [(jnp.maximum(iota - d, 0), iota >= d)
                        for d in (1, 2, 4, 8)]
        in_sems = (in_sem0, in_sem1)
        out_sems = (out_sem0, out_sem1)

        def row_idx(r):
            return wid * ROWS_PER_WORKER + r

        def scan_row(b):
            def group_body(g, c):
                base = g * (UNROLL * LANES)
                sls = [pl.ds(base + j * LANES, LANES) for j in range(UNROLL)]
                ss = [plsc.cumsum(buf[b, sl]) for sl in sls]
                ts = [s.at[idx_last].get(mode="promise_in_bounds")
                      for s in ss]
                incl = _inclusive_prefix_tree(ts)
                pres = [c] + [c + incl[j] for j in range(UNROLL - 1)]
                for j in range(UNROLL):
                    buf[b, sls[j]] = ss[j] + pres[j]
                return c + incl[UNROLL - 1]

            plsc.parallel_loop(
                0, NGROUPS, 1, carry=jnp.zeros((LANES,), jnp.float32)
            )(group_body)

        pending_out = [None, None]
        copy_in = pltpu.async_copy(
            x_hbm.at[row_idx(0)], buf.at[0], in_sems[0])
        for r in range(ROWS_PER_WORKER):
            b = r % 2
            nb = (r + 1) % 2
            if r + 1 < ROWS_PER_WORKER:
                if pending_out[nb] is not None:
                    pending_out[nb].wait()
                    pending_out[nb] = None
                next_in = pltpu.async_copy(
                    x_hbm.at[row_idx(r + 1)], buf.at[nb], in_sems[nb])
            copy_in.wait()
            scan_row(b)
            pending_out[b] = pltpu.async_copy(
                buf.at[b], out_hbm.at[row_idx(r)], out_sems[b])
            if r + 1 < ROWS_PER_WORKER:
                copy_in = next_in
        for p in pending_out:
            if p is not None:
                p.wait()

    return k(x)


def kernel(x):
    return _sc_row_cumsum(x)
